# Initial kernel scaffold; baseline (speedup 1.0000x reference)
import jax
import jax.numpy as jnp
from jax.experimental import pallas as pl


def kernel(node2_features, mpnn_features, edge_index_r0, edge_index_r1, dec_edge_index, sage2_W1, sage2_b1, sage2_W2, sage2_b2, sage3_W1, sage3_b1, sage3_W2, sage3_b2, pred_W1, pred_b1, pred_W2, pred_b2):
    raise NotImplementedError("write your pallas kernel here")


# speedup vs baseline: 4.2814x; 4.2814x over previous
import functools

import jax
import jax.numpy as jnp
from jax import lax
from jax.experimental import pallas as pl
from jax.experimental.pallas import tpu as pltpu
from jax.experimental.pallas import tpu_sc as plsc

N = 10000
E = 160000
D = 128
NC, NS, L = 2, 16, 16
EPT = E // NS
WB = 1000
NWB = N // WB
CH = 128
NFULL = EPT // CH
REM = EPT - NFULL * CH
EPW = E // (NC * NS)
NFULL_P = EPW // CH
REM_P = EPW - NFULL_P * CH
TW = 16

_mesh = plsc.VectorSubcoreMesh(core_axis_name="c", subcore_axis_name="s")


def _tc_l1_body(x2_ref, x3_ref, w2_ref, w3_ref, t0_ref, t1_ref):
    x2 = x2_ref[...]
    x3 = x3_ref[...]
    for r, t_ref in ((0, t0_ref), (1, t1_ref)):
        t_ref[:N, :] = jnp.dot(x2, w2_ref[r], preferred_element_type=jnp.float32)
        t_ref[N:, :] = jnp.dot(x3, w3_ref[r], preferred_element_type=jnp.float32)


def _tc_l1(x2, x3, w2, w3):
    return pl.pallas_call(
        _tc_l1_body,
        out_shape=[jax.ShapeDtypeStruct((2 * N, D), jnp.float32)] * 2,
        compiler_params=pltpu.CompilerParams(vmem_limit_bytes=100 * 1024 * 1024),
    )(x2, x3, w2, w3)


def _tc_l2_body(a0_ref, a1_ref, d0_ref, d1_ref, b2_ref, b3_ref,
                w2_ref, w3_ref, t0_ref, t1_ref):
    inv0 = 1.0 / jnp.maximum(d0_ref[...], 1.0)
    inv1 = 1.0 / jnp.maximum(d1_ref[...], 1.0)
    h2 = jax.nn.relu(a0_ref[:N, :] * inv0 + b2_ref[0] +
                     a1_ref[:N, :] * inv1 + b2_ref[1])
    h3 = jax.nn.relu(a0_ref[N:, :] * inv0 + b3_ref[0] +
                     a1_ref[N:, :] * inv1 + b3_ref[1])
    for r, t_ref in ((0, t0_ref), (1, t1_ref)):
        t_ref[:N, :] = jnp.dot(h2, w2_ref[r], preferred_element_type=jnp.float32)
        t_ref[N:, :] = jnp.dot(h3, w3_ref[r], preferred_element_type=jnp.float32)


def _tc_l2(a0, a1, d0, d1, b2, b3, w2, w3):
    return pl.pallas_call(
        _tc_l2_body,
        out_shape=[jax.ShapeDtypeStruct((2 * N, D), jnp.float32)] * 2,
        compiler_params=pltpu.CompilerParams(vmem_limit_bytes=100 * 1024 * 1024),
    )(a0, a1, d0, d1, b2, b3, w2, w3)


def _tc_pred_body(a0_ref, a1_ref, d0_ref, d1_ref, b2_ref, b3_ref,
                  pw1_ref, pb1_ref, pw2_ref, pb2_ref, ta_ref, tb_ref):
    inv0 = 1.0 / jnp.maximum(d0_ref[...], 1.0)
    inv1 = 1.0 / jnp.maximum(d1_ref[...], 1.0)
    h2 = (a0_ref[:N, :] * inv0 + b2_ref[0] +
          a1_ref[:N, :] * inv1 + b2_ref[1])
    h3 = (a0_ref[N:, :] * inv0 + b3_ref[0] +
          a1_ref[N:, :] * inv1 + b3_ref[1])
    m = jnp.dot(pw1_ref[...], pw2_ref[...], preferred_element_type=jnp.float32)
    cb = jnp.dot(pb1_ref[...].reshape(1, -1), pw2_ref[...],
                 preferred_element_type=jnp.float32) + pb2_ref[...].reshape(1, -1)
    a2 = (jnp.dot(h2, m[0:D], preferred_element_type=jnp.float32) +
          jnp.dot(h3, m[D:2 * D], preferred_element_type=jnp.float32) + cb)
    b2t = (jnp.dot(h2, m[2 * D:3 * D], preferred_element_type=jnp.float32) +
           jnp.dot(h3, m[3 * D:4 * D], preferred_element_type=jnp.float32))
    ta_ref[...] = a2
    tb_ref[...] = b2t


def _tc_pred(a0, a1, d0, d1, b2, b3, pw1, pb1, pw2, pb2):
    return pl.pallas_call(
        _tc_pred_body,
        out_shape=[jax.ShapeDtypeStruct((N, 2), jnp.float32)] * 2,
        compiler_params=pltpu.CompilerParams(vmem_limit_bytes=100 * 1024 * 1024),
    )(a0, a1, d0, d1, b2, b3, pw1, pb1, pw2, pb2)


def _sc_agg_body(with_deg, *refs):
    if with_deg:
        (t0_hbm, t1_hbm, s0_hbm, d0_hbm, s1_hbm, d1_hbm, z_hbm, zd_hbm,
         a0_hbm, a1_hbm, g0_hbm, g1_hbm,
         sidx_v, didx_v, rows_v, ones_v, acc_sh, dega_sh, gsem) = refs
    else:
        (t0_hbm, t1_hbm, s0_hbm, d0_hbm, s1_hbm, d1_hbm, z_hbm,
         a0_hbm, a1_hbm,
         sidx_v, didx_v, rows_v, acc_sh, gsem) = refs
    c = lax.axis_index("c")
    s = lax.axis_index("s")
    coff = c * N
    ebase = s * EPT

    if with_deg:
        def _init_ones(i, carry):
            ones_v[pl.ds(i * L, L)] = jnp.ones((L,), jnp.float32)
            return carry
        lax.fori_loop(0, CH // L, _init_ones, 0)

    for r, (t_hbm, src_hbm, dst_hbm, agg_hbm) in enumerate(
            ((t0_hbm, s0_hbm, d0_hbm, a0_hbm), (t1_hbm, s1_hbm, d1_hbm, a1_hbm))):
        @pl.when(s < NWB)
        def _():
            pltpu.sync_copy(z_hbm, acc_sh.at[pl.ds(s * WB, WB)])
        if with_deg:
            @pl.when(jnp.logical_and(c == r, s == 0))
            def _():
                pltpu.sync_copy(zd_hbm, dega_sh)
        plsc.subcore_barrier()

        def _chunk(k, carry, sz=CH):
            eo = pl.multiple_of(ebase + k * CH, 8)
            pltpu.sync_copy(src_hbm.at[pl.ds(eo, sz)], sidx_v.at[pl.ds(0, sz)])

            def _adj(i, cc):
                sidx_v[pl.ds(i * L, L)] = sidx_v[pl.ds(i * L, L)] + coff
                return cc
            lax.fori_loop(0, sz // L, _adj, 0)
            pltpu.async_copy(t_hbm.at[sidx_v.at[pl.ds(0, sz)]],
                             rows_v.at[pl.ds(0, sz)], gsem).wait()
            pltpu.sync_copy(dst_hbm.at[pl.ds(eo, sz)], didx_v.at[pl.ds(0, sz)])
            pltpu.sync_copy(rows_v.at[pl.ds(0, sz)],
                            acc_sh.at[didx_v.at[pl.ds(0, sz)]], add=True)
            if with_deg:
                @pl.when(c == r)
                def _():
                    pltpu.sync_copy(ones_v.at[pl.ds(0, sz)],
                                    dega_sh.at[didx_v.at[pl.ds(0, sz)]], add=True)
            return carry
        lax.fori_loop(0, NFULL, _chunk, 0)
        _chunk(NFULL, 0, sz=REM)

        plsc.subcore_barrier()

        @pl.when(s < NWB)
        def _():
            pltpu.sync_copy(acc_sh.at[pl.ds(s * WB, WB)],
                            agg_hbm.at[pl.ds(coff + s * WB, WB)])
        if with_deg:
            @pl.when(jnp.logical_and(c == r, s == 0))
            def _():
                pltpu.sync_copy(dega_sh, g0_hbm if r == 0 else g1_hbm)
        plsc.subcore_barrier()


def _sc_agg_deg(t0, t1, s0, d0, s1, d1, z, zd):
    f = pl.kernel(
        functools.partial(_sc_agg_body, True),
        out_type=[jax.ShapeDtypeStruct((2 * N, D), jnp.float32)] * 2 +
                 [jax.ShapeDtypeStruct((N,), jnp.float32)] * 2,
        mesh=_mesh,
        scratch_types=[
            pltpu.VMEM((CH,), jnp.int32),
            pltpu.VMEM((CH,), jnp.int32),
            pltpu.VMEM((CH, D), jnp.float32),
            pltpu.VMEM((CH,), jnp.float32),
            pltpu.VMEM_SHARED((N, D), jnp.float32),
            pltpu.VMEM_SHARED((N,), jnp.float32),
            pltpu.SemaphoreType.DMA,
        ],
    )
    return f(t0, t1, s0, d0, s1, d1, z, zd)


def _sc_agg(t0, t1, s0, d0, s1, d1, z):
    f = pl.kernel(
        functools.partial(_sc_agg_body, False),
        out_type=[jax.ShapeDtypeStruct((2 * N, D), jnp.float32)] * 2,
        mesh=_mesh,
        scratch_types=[
            pltpu.VMEM((CH,), jnp.int32),
            pltpu.VMEM((CH,), jnp.int32),
            pltpu.VMEM((CH, D), jnp.float32),
            pltpu.VMEM_SHARED((N, D), jnp.float32),
            pltpu.SemaphoreType.DMA,
        ],
    )
    return f(t0, t1, s0, d0, s1, d1, z)


def _sc_pred_body(ta_hbm, tb_hbm, ds_hbm, dd_hbm, out_hbm,
                  ta_v, tb_v, sidx_v, didx_v, ov_v):
    c = lax.axis_index("c")
    s = lax.axis_index("s")
    wid = s * NC + c
    base = pl.multiple_of(wid * EPW, 8)
    pltpu.sync_copy(ta_hbm, ta_v)
    pltpu.sync_copy(tb_hbm, tb_v)
    pltpu.sync_copy(ds_hbm.at[pl.ds(base, EPW)], sidx_v.at[pl.ds(0, EPW)])
    pltpu.sync_copy(dd_hbm.at[pl.ds(base, EPW)], didx_v.at[pl.ds(0, EPW)])

    lanes = lax.iota(jnp.int32, L)

    def _group(eoff):
        si = sidx_v[pl.ds(eoff, L)] * 2
        di = didx_v[pl.ds(eoff, L)] * 2
        s0 = plsc.load_gather(ta_v, [si]) + plsc.load_gather(tb_v, [di])
        s1 = plsc.load_gather(ta_v, [si + 1]) + plsc.load_gather(tb_v, [di + 1])
        oi = eoff * 2 + lanes * 2
        plsc.store_scatter(ov_v, [oi], s0)
        plsc.store_scatter(ov_v, [oi + 1], s1)

    def _body(g, carry):
        _group(g * L)
        return carry
    lax.fori_loop(0, EPW // L, _body, 0)
    if EPW % L:
        _group(EPW - L)
    pltpu.sync_copy(ov_v, out_hbm.at[pl.ds(base * 2, 2 * EPW)])


def _sc_pred(ta, tb, dsrc, ddst):
    idx_pad = EPW
    f = pl.kernel(
        _sc_pred_body,
        out_type=jax.ShapeDtypeStruct((2 * E,), jnp.float32),
        mesh=_mesh,
        scratch_types=[
            pltpu.VMEM((2 * N,), jnp.float32),
            pltpu.VMEM((2 * N,), jnp.float32),
            pltpu.VMEM((idx_pad,), jnp.int32),
            pltpu.VMEM((idx_pad,), jnp.int32),
            pltpu.VMEM((2 * EPW,), jnp.float32),
        ],
        compiler_params=pltpu.CompilerParams(needs_layout_passes=False),
    )
    return f(ta, tb, dsrc, ddst)


def kernel(node2_features, mpnn_features, edge_index_r0, edge_index_r1,
           dec_edge_index,
           sage2_W1, sage2_b1, sage2_W2, sage2_b2,
           sage3_W1, sage3_b1, sage3_W2, sage3_b2,
           pred_W1, pred_b1, pred_W2, pred_b2):
    s0, d0 = edge_index_r0[0], edge_index_r0[1]
    s1, d1 = edge_index_r1[0], edge_index_r1[1]
    dsrc, ddst = dec_edge_index[0], dec_edge_index[1]
    z = jnp.zeros((WB, D), jnp.float32)
    zd = jnp.zeros((N,), jnp.float32)

    t10, t11 = _tc_l1(node2_features, mpnn_features, sage2_W1, sage3_W1)
    a10, a11, deg0, deg1 = _sc_agg_deg(t10, t11, s0, d0, s1, d1, z, zd)
    d0c = deg0.reshape(N, 1)
    d1c = deg1.reshape(N, 1)
    t20, t21 = _tc_l2(a10, a11, d0c, d1c, sage2_b1, sage3_b1,
                      sage2_W2, sage3_W2)
    a20, a21 = _sc_agg(t20, t21, s0, d0, s1, d1, z)
    ta, tb = _tc_pred(a20, a21, d0c, d1c, sage2_b2, sage3_b2,
                      pred_W1, pred_b1, pred_W2, pred_b2)
    outp = _sc_pred(ta.reshape(-1), tb.reshape(-1), dsrc, ddst)
    return outp.reshape(E, 2)
